# baseline (speedup 1.0000x reference)
import dataclasses
import functools

import jax
import jax.numpy as jnp
from jax import lax
from jax.experimental import pallas as pl
from jax.experimental.pallas import tpu as pltpu
from jax.experimental.pallas import tpu_sc as plsc

N = 10000
E = 320000
D = 128
NC = 2
NS = 16
NW = NC * NS
CW = 128
NCH = 80
EPT = NCH * CW
EPAD = NW * EPT
WTILES = 10
WROWS = N // WTILES

_mesh = plsc.VectorSubcoreMesh(core_axis_name="c", subcore_axis_name="s")

_sc_params = pltpu.CompilerParams()
if "needs_layout_passes" in pltpu.CompilerParams.__dataclass_fields__:
    _sc_params = dataclasses.replace(_sc_params, needs_layout_passes=False)


def _deg_call(col3, ew1):
    @functools.partial(
        pl.kernel,
        out_type=jax.ShapeDtypeStruct((NC, N), jnp.float32),
        mesh=_mesh,
        compiler_params=_sc_params,
        scratch_types=[
            pltpu.VMEM((NCH, CW), jnp.int32),
            pltpu.VMEM((EPT,), jnp.float32),
            pltpu.VMEM_SHARED((N,), jnp.float32),
        ],
    )
    def k(col_hbm, ew_hbm, z_hbm, out_hbm, col_v, ew_v, deg_sh):
        cid = lax.axis_index("c")
        sid = lax.axis_index("s")
        wid = cid * NS + sid

        @pl.when(sid == 0)
        def _init():
            pltpu.sync_copy(z_hbm, deg_sh)

        pltpu.sync_copy(col_hbm.at[wid], col_v)
        pltpu.sync_copy(ew_hbm.at[wid], ew_v)
        plsc.subcore_barrier()

        @pl.loop(0, NCH)
        def _chunk(j):
            off = pl.multiple_of(j * CW, 8)
            pltpu.sync_copy(ew_v.at[pl.ds(off, CW)],
                            deg_sh.at[col_v.at[j]], add=True)

        plsc.subcore_barrier()

        @pl.when(sid == 0)
        def _writeout():
            pltpu.sync_copy(deg_sh, out_hbm.at[cid])

    return k(col3, ew1, jnp.zeros((N,), jnp.float32))


NCH0 = 28
NCH1 = 130
NCHMAX = max(NCH0, NCH1)
TOT0 = NS * NCH0 * CW
TOT1 = NS * NCH1 * CW


def _agg_call(hpp, row1, col3, ew1):
    @functools.partial(
        pl.kernel,
        out_type=jax.ShapeDtypeStruct((NC, N, D), jnp.float32),
        mesh=_mesh,
        compiler_params=_sc_params,
        scratch_types=[
            pltpu.VMEM((NCHMAX, CW), jnp.int32),
            pltpu.VMEM((CW,), jnp.int32),
            pltpu.VMEM((CW,), jnp.int32),
            pltpu.VMEM((CW,), jnp.float32),
            pltpu.VMEM((CW,), jnp.float32),
            pltpu.VMEM((CW, D), jnp.float32),
            pltpu.VMEM((CW, D), jnp.float32),
            pltpu.VMEM_SHARED((N, D), jnp.float32),
            pltpu.SemaphoreType.DMA,
            pltpu.SemaphoreType.DMA,
            pltpu.SemaphoreType.DMA,
            pltpu.SemaphoreType.DMA,
            pltpu.SemaphoreType.DMA,
            pltpu.SemaphoreType.DMA,
        ],
    )
    def k(hpp_hbm, row_hbm, col_hbm, ew_hbm, z_hbm, out_hbm,
          col_v, rw0, rw1, eb0, eb1, b0, b1, acc_sh,
          g0, g1, s0, s1, rsem, esem):
        cid = lax.axis_index("c")
        sid = lax.axis_index("s")
        wid = cid * NS + sid
        bufs = (b0, b1)
        rws = (rw0, rw1)
        ebs = (eb0, eb1)
        gsems = (g0, g1)
        ssems = (s0, s1)
        nch = jnp.where(cid == 0, NCH0, NCH1)
        ebase = jnp.where(cid == 0, sid * (NCH0 * CW),
                          TOT0 + sid * (NCH1 * CW))

        def r_fetch(w, b):
            off = pl.multiple_of(ebase + w * CW, 8)
            pltpu.async_copy(row_hbm.at[pl.ds(off, CW)], rws[b], rsem)

        def r_wait(b):
            pltpu.make_async_copy(row_hbm.at[pl.ds(0, CW)], rws[b],
                                  rsem).wait()

        def e_fetch(w, b):
            off = pl.multiple_of(ebase + w * CW, 8)
            pltpu.async_copy(ew_hbm.at[pl.ds(off, CW)], ebs[b], esem)

        def e_wait(b):
            pltpu.make_async_copy(ew_hbm.at[pl.ds(0, CW)], ebs[b],
                                  esem).wait()

        def g_start(b):
            pltpu.async_copy(hpp_hbm.at[rws[b]], bufs[b], gsems[b])

        def g_wait(b):
            pltpu.make_async_copy(hpp_hbm.at[rws[b]], bufs[b],
                                  gsems[b]).wait()

        def s_start(w, b):
            pltpu.async_copy(bufs[b], acc_sh.at[col_v.at[w]], ssems[b],
                             add=True)

        def s_wait(b):
            pltpu.make_async_copy(bufs[b], acc_sh.at[col_v.at[0]],
                                  ssems[b]).wait()

        def scale(b):
            buf = bufs[b]
            ewb = ebs[b]

            @pl.loop(0, CW // 16)
            def _grp(g):
                off16 = pl.multiple_of(g * 16, 8)
                ew16 = ewb[pl.ds(off16, 16)]
                for t in range(16):
                    sp = lax.gather(
                        ew16, jnp.full((16, 1), t, jnp.int32),
                        dimension_numbers=lax.GatherDimensionNumbers(
                            offset_dims=(), collapsed_slice_dims=(0,),
                            start_index_map=(0,)),
                        slice_sizes=(1,),
                        mode=lax.GatherScatterMode.PROMISE_IN_BOUNDS)
                    for q in range(D // 16):
                        sl = pl.ds(q * 16, 16)
                        buf[g * 16 + t, sl] = buf[g * 16 + t, sl] * sp

        pltpu.sync_copy(col_hbm.at[wid], col_v)
        off0 = pl.multiple_of(ebase, 8)
        off1 = pl.multiple_of(ebase + CW, 8)
        pltpu.sync_copy(row_hbm.at[pl.ds(off0, CW)], rw0)
        pltpu.sync_copy(row_hbm.at[pl.ds(off1, CW)], rw1)
        pltpu.sync_copy(ew_hbm.at[pl.ds(off0, CW)], eb0)
        pltpu.sync_copy(ew_hbm.at[pl.ds(off1, CW)], eb1)

        @pl.when(sid < WTILES)
        def _init():
            base = pl.multiple_of(sid * WROWS, 8)
            pltpu.sync_copy(z_hbm, acc_sh.at[pl.ds(base, WROWS)])

        g_start(0)
        plsc.subcore_barrier()

        @pl.loop(0, nch, step=2)
        def _pipe(j):
            for t in range(2):
                w = j + t
                b = t
                bp = 1 - t
                g_wait(b)
                r_fetch(jnp.minimum(w + 2, nch - 1), b)
                if t == 0:
                    @pl.when(j > 0)
                    def _swrw():
                        s_wait(bp)
                        r_wait(bp)
                else:
                    s_wait(bp)
                    r_wait(bp)
                g_start(bp)
                if t == 0:
                    @pl.when(j > 0)
                    def _ew():
                        e_wait(b)
                else:
                    @pl.when(j > 0)
                    def _ew2():
                        e_wait(b)
                scale(b)
                e_fetch(jnp.minimum(w + 2, nch - 1), b)
                s_start(w, b)

        g_wait(0)
        s_wait(1)
        r_wait(0)
        e_wait(0)
        e_wait(1)
        plsc.subcore_barrier()

        @pl.when(sid < WTILES)
        def _writeout():
            base = pl.multiple_of(sid * WROWS, 8)
            pltpu.sync_copy(acc_sh.at[pl.ds(base, WROWS)],
                            out_hbm.at[cid, pl.ds(base, WROWS)])

    return k(hpp, row1, col3, ew1, jnp.zeros((WROWS, D), jnp.float32))


def _tc_pre_call(x, W1, degp):
    def body(x_ref, w_ref, degp_ref, hpp_ref, dinv_ref):
        deg = degp_ref[0] + degp_ref[1] + 1.0
        dinv = jnp.where(deg > 0, lax.rsqrt(deg), 0.0)
        h = jnp.dot(x_ref[...], w_ref[...],
                    preferred_element_type=jnp.float32,
                    precision=lax.Precision.HIGHEST)
        hpp_ref[...] = h * dinv
        dinv_ref[...] = dinv

    return pl.pallas_call(
        body,
        out_shape=(jax.ShapeDtypeStruct((N, D), jnp.float32),
                   jax.ShapeDtypeStruct((N, 1), jnp.float32)),
    )(x, W1, degp)


def _tc_mid_call(accp, hpp, dinv, b2d, Wn):
    def body(accp_ref, hpp_ref, dinv_ref, b_ref, w_ref, out_ref):
        s = accp_ref[0] + accp_ref[1] + hpp_ref[...]
        y = jnp.maximum(dinv_ref[...] * s + b_ref[...], 0.0)
        h = jnp.dot(y, w_ref[...],
                    preferred_element_type=jnp.float32,
                    precision=lax.Precision.HIGHEST)
        out_ref[...] = h * dinv_ref[...]

    return pl.pallas_call(
        body,
        out_shape=jax.ShapeDtypeStruct((N, D), jnp.float32),
    )(accp, hpp, dinv, b2d, Wn)


def _tc_final_call(accp, hpp, dinv, b2d):
    def body(accp_ref, hpp_ref, dinv_ref, b_ref, out_ref):
        s = accp_ref[0] + accp_ref[1] + hpp_ref[...]
        out_ref[...] = jnp.maximum(dinv_ref[...] * s + b_ref[...], 0.0)

    return pl.pallas_call(
        body,
        out_shape=jax.ShapeDtypeStruct((N, D), jnp.float32),
    )(accp, hpp, dinv, b2d)


def kernel(x, edge_index, edge_weights, W1, b1, W2, b2, W3, b3):
    padd = EPAD - E
    dcol3 = jnp.pad(edge_index[1], (0, padd)).reshape(NW, NCH, CW)
    dew1 = jnp.pad(edge_weights, (0, padd)).reshape(NW, EPT)

    pada = TOT0 + TOT1 - E
    row1 = jnp.pad(edge_index[0], (0, pada))
    ewf = jnp.pad(edge_weights, (0, pada))
    colf = jnp.pad(edge_index[1], (0, pada))
    c0 = jnp.pad(colf[:TOT0].reshape(NS, NCH0, CW),
                 ((0, 0), (0, NCHMAX - NCH0), (0, 0)))
    c1 = jnp.pad(colf[TOT0:].reshape(NS, NCH1, CW),
                 ((0, 0), (0, NCHMAX - NCH1), (0, 0)))
    col3 = jnp.concatenate([c0, c1], axis=0)

    degp = _deg_call(dcol3, dew1)[:, :, None]
    hpp1, dinv = _tc_pre_call(x, W1, degp)
    acc1 = _agg_call(hpp1, row1, col3, ewf)
    hpp2 = _tc_mid_call(acc1, hpp1, dinv, b1.reshape(1, D), W2)
    acc2 = _agg_call(hpp2, row1, col3, ewf)
    hpp3 = _tc_mid_call(acc2, hpp2, dinv, b2.reshape(1, D), W3)
    acc3 = _agg_call(hpp3, row1, col3, ewf)
    return _tc_final_call(acc3, hpp3, dinv, b3.reshape(1, D))


# speedup vs baseline: 1.0501x; 1.0501x over previous
import dataclasses
import functools

import jax
import jax.numpy as jnp
from jax import lax
from jax.experimental import pallas as pl
from jax.experimental.pallas import tpu as pltpu
from jax.experimental.pallas import tpu_sc as plsc

N = 10000
E = 320000
D = 128
NC = 2
NS = 16
NW = NC * NS
CW = 128
NCH = 80
EPT = NCH * CW
EPAD = NW * EPT
WTILES = 10
WROWS = N // WTILES

_mesh = plsc.VectorSubcoreMesh(core_axis_name="c", subcore_axis_name="s")

_sc_params = pltpu.CompilerParams()
if "needs_layout_passes" in pltpu.CompilerParams.__dataclass_fields__:
    _sc_params = dataclasses.replace(_sc_params, needs_layout_passes=False)


def _deg_call(col3, ew1):
    @functools.partial(
        pl.kernel,
        out_type=jax.ShapeDtypeStruct((NC, N), jnp.float32),
        mesh=_mesh,
        compiler_params=_sc_params,
        scratch_types=[
            pltpu.VMEM((NCH, CW), jnp.int32),
            pltpu.VMEM((EPT,), jnp.float32),
            pltpu.VMEM_SHARED((N,), jnp.float32),
        ],
    )
    def k(col_hbm, ew_hbm, z_hbm, out_hbm, col_v, ew_v, deg_sh):
        cid = lax.axis_index("c")
        sid = lax.axis_index("s")
        wid = cid * NS + sid

        @pl.when(sid == 0)
        def _init():
            pltpu.sync_copy(z_hbm, deg_sh)

        pltpu.sync_copy(col_hbm.at[wid], col_v)
        pltpu.sync_copy(ew_hbm.at[wid], ew_v)
        plsc.subcore_barrier()

        @pl.loop(0, NCH)
        def _chunk(j):
            off = pl.multiple_of(j * CW, 8)
            pltpu.sync_copy(ew_v.at[pl.ds(off, CW)],
                            deg_sh.at[col_v.at[j]], add=True)

        plsc.subcore_barrier()

        @pl.when(sid == 0)
        def _writeout():
            pltpu.sync_copy(deg_sh, out_hbm.at[cid])

    return k(col3, ew1, jnp.zeros((N,), jnp.float32))


NCH0 = 32
NCH1 = 126
NCHMAX = max(NCH0, NCH1)
TOT0 = NS * NCH0 * CW
TOT1 = NS * NCH1 * CW


def _agg_call(hpp, row1, col3, ew1):
    @functools.partial(
        pl.kernel,
        out_type=jax.ShapeDtypeStruct((NC, N, D), jnp.float32),
        mesh=_mesh,
        compiler_params=_sc_params,
        scratch_types=[
            pltpu.VMEM((NCHMAX, CW), jnp.int32),
            pltpu.VMEM((CW,), jnp.int32),
            pltpu.VMEM((CW,), jnp.int32),
            pltpu.VMEM((CW,), jnp.float32),
            pltpu.VMEM((CW,), jnp.float32),
            pltpu.VMEM((CW, D), jnp.float32),
            pltpu.VMEM((CW, D), jnp.float32),
            pltpu.VMEM_SHARED((N, D), jnp.float32),
            pltpu.SemaphoreType.DMA,
            pltpu.SemaphoreType.DMA,
            pltpu.SemaphoreType.DMA,
            pltpu.SemaphoreType.DMA,
            pltpu.SemaphoreType.DMA,
            pltpu.SemaphoreType.DMA,
        ],
    )
    def k(hpp_hbm, row_hbm, col_hbm, ew_hbm, z_hbm, out_hbm,
          col_v, rw0, rw1, eb0, eb1, b0, b1, acc_sh,
          g0, g1, s0, s1, rsem, esem):
        cid = lax.axis_index("c")
        sid = lax.axis_index("s")
        wid = cid * NS + sid
        bufs = (b0, b1)
        rws = (rw0, rw1)
        ebs = (eb0, eb1)
        gsems = (g0, g1)
        ssems = (s0, s1)
        nch = jnp.where(cid == 0, NCH0, NCH1)
        ebase = jnp.where(cid == 0, sid * (NCH0 * CW),
                          TOT0 + sid * (NCH1 * CW))

        def r_fetch(w, b):
            off = pl.multiple_of(ebase + w * CW, 8)
            pltpu.async_copy(row_hbm.at[pl.ds(off, CW)], rws[b], rsem)

        def r_wait(b):
            pltpu.make_async_copy(row_hbm.at[pl.ds(0, CW)], rws[b],
                                  rsem).wait()

        def e_fetch(w, b):
            off = pl.multiple_of(ebase + w * CW, 8)
            pltpu.async_copy(ew_hbm.at[pl.ds(off, CW)], ebs[b], esem)

        def e_wait(b):
            pltpu.make_async_copy(ew_hbm.at[pl.ds(0, CW)], ebs[b],
                                  esem).wait()

        def g_start(b):
            pltpu.async_copy(hpp_hbm.at[rws[b]], bufs[b], gsems[b])

        def g_wait(b):
            pltpu.make_async_copy(hpp_hbm.at[rws[b]], bufs[b],
                                  gsems[b]).wait()

        def s_start(w, b):
            pltpu.async_copy(bufs[b], acc_sh.at[col_v.at[w]], ssems[b],
                             add=True)

        def s_wait(b):
            pltpu.make_async_copy(bufs[b], acc_sh.at[col_v.at[0]],
                                  ssems[b]).wait()

        def scale(b):
            buf = bufs[b]
            ewb = ebs[b]

            @pl.loop(0, CW // 16)
            def _grp(g):
                off16 = pl.multiple_of(g * 16, 8)
                ew16 = ewb[pl.ds(off16, 16)]
                for t in range(16):
                    sp = lax.gather(
                        ew16, jnp.full((16, 1), t, jnp.int32),
                        dimension_numbers=lax.GatherDimensionNumbers(
                            offset_dims=(), collapsed_slice_dims=(0,),
                            start_index_map=(0,)),
                        slice_sizes=(1,),
                        mode=lax.GatherScatterMode.PROMISE_IN_BOUNDS)
                    for q in range(D // 16):
                        sl = pl.ds(q * 16, 16)
                        buf[g * 16 + t, sl] = buf[g * 16 + t, sl] * sp

        pltpu.sync_copy(col_hbm.at[wid], col_v)
        off0 = pl.multiple_of(ebase, 8)
        off1 = pl.multiple_of(ebase + CW, 8)
        pltpu.sync_copy(row_hbm.at[pl.ds(off0, CW)], rw0)
        pltpu.sync_copy(row_hbm.at[pl.ds(off1, CW)], rw1)
        pltpu.sync_copy(ew_hbm.at[pl.ds(off0, CW)], eb0)
        pltpu.sync_copy(ew_hbm.at[pl.ds(off1, CW)], eb1)

        @pl.when(sid < WTILES)
        def _init():
            base = pl.multiple_of(sid * WROWS, 8)
            pltpu.sync_copy(z_hbm, acc_sh.at[pl.ds(base, WROWS)])

        g_start(0)
        plsc.subcore_barrier()

        @pl.loop(0, nch, step=2)
        def _pipe(j):
            for t in range(2):
                w = j + t
                b = t
                bp = 1 - t
                g_wait(b)
                r_fetch(jnp.minimum(w + 2, nch - 1), b)
                if t == 0:
                    @pl.when(j > 0)
                    def _swrw():
                        s_wait(bp)
                        r_wait(bp)
                else:
                    s_wait(bp)
                    r_wait(bp)
                g_start(bp)
                if t == 0:
                    @pl.when(j > 0)
                    def _ew():
                        e_wait(b)
                else:
                    @pl.when(j > 0)
                    def _ew2():
                        e_wait(b)
                scale(b)
                e_fetch(jnp.minimum(w + 2, nch - 1), b)
                s_start(w, b)

        g_wait(0)
        s_wait(1)
        r_wait(0)
        e_wait(0)
        e_wait(1)
        plsc.subcore_barrier()

        @pl.when(sid < WTILES)
        def _writeout():
            base = pl.multiple_of(sid * WROWS, 8)
            pltpu.sync_copy(acc_sh.at[pl.ds(base, WROWS)],
                            out_hbm.at[cid, pl.ds(base, WROWS)])

    return k(hpp, row1, col3, ew1, jnp.zeros((WROWS, D), jnp.float32))


def _tc_pre_call(x, W1, degp):
    def body(x_ref, w_ref, degp_ref, hpp_ref, dinv_ref):
        deg = degp_ref[0] + degp_ref[1] + 1.0
        dinv = jnp.where(deg > 0, lax.rsqrt(deg), 0.0)
        h = jnp.dot(x_ref[...], w_ref[...],
                    preferred_element_type=jnp.float32,
                    precision=lax.Precision.HIGHEST)
        hpp_ref[...] = h * dinv
        dinv_ref[...] = dinv

    return pl.pallas_call(
        body,
        out_shape=(jax.ShapeDtypeStruct((N, D), jnp.float32),
                   jax.ShapeDtypeStruct((N, 1), jnp.float32)),
    )(x, W1, degp)


def _tc_mid_call(accp, hpp, dinv, b2d, Wn):
    def body(accp_ref, hpp_ref, dinv_ref, b_ref, w_ref, out_ref):
        s = accp_ref[0] + accp_ref[1] + hpp_ref[...]
        y = jnp.maximum(dinv_ref[...] * s + b_ref[...], 0.0)
        h = jnp.dot(y, w_ref[...],
                    preferred_element_type=jnp.float32,
                    precision=lax.Precision.HIGHEST)
        out_ref[...] = h * dinv_ref[...]

    return pl.pallas_call(
        body,
        out_shape=jax.ShapeDtypeStruct((N, D), jnp.float32),
    )(accp, hpp, dinv, b2d, Wn)


def _tc_final_call(accp, hpp, dinv, b2d):
    def body(accp_ref, hpp_ref, dinv_ref, b_ref, out_ref):
        s = accp_ref[0] + accp_ref[1] + hpp_ref[...]
        out_ref[...] = jnp.maximum(dinv_ref[...] * s + b_ref[...], 0.0)

    return pl.pallas_call(
        body,
        out_shape=jax.ShapeDtypeStruct((N, D), jnp.float32),
    )(accp, hpp, dinv, b2d)


def kernel(x, edge_index, edge_weights, W1, b1, W2, b2, W3, b3):
    padd = EPAD - E
    dcol3 = jnp.pad(edge_index[1], (0, padd)).reshape(NW, NCH, CW)
    dew1 = jnp.pad(edge_weights, (0, padd)).reshape(NW, EPT)

    pada = TOT0 + TOT1 - E
    row1 = jnp.pad(edge_index[0], (0, pada))
    ewf = jnp.pad(edge_weights, (0, pada))
    colf = jnp.pad(edge_index[1], (0, pada))
    c0 = jnp.pad(colf[:TOT0].reshape(NS, NCH0, CW),
                 ((0, 0), (0, NCHMAX - NCH0), (0, 0)))
    c1 = jnp.pad(colf[TOT0:].reshape(NS, NCH1, CW),
                 ((0, 0), (0, NCHMAX - NCH1), (0, 0)))
    col3 = jnp.concatenate([c0, c1], axis=0)

    degp = _deg_call(dcol3, dew1)[:, :, None]
    hpp1, dinv = _tc_pre_call(x, W1, degp)
    acc1 = _agg_call(hpp1, row1, col3, ewf)
    hpp2 = _tc_mid_call(acc1, hpp1, dinv, b1.reshape(1, D), W2)
    acc2 = _agg_call(hpp2, row1, col3, ewf)
    hpp3 = _tc_mid_call(acc2, hpp2, dinv, b2.reshape(1, D), W3)
    acc3 = _agg_call(hpp3, row1, col3, ewf)
    return _tc_final_call(acc3, hpp3, dinv, b3.reshape(1, D))
